# Initial kernel scaffold; baseline (speedup 1.0000x reference)
import jax
import jax.numpy as jnp
from jax.experimental import pallas as pl


def kernel(features, edge_attr, params, edge_index):
    raise NotImplementedError("write your pallas kernel here")


# speedup vs baseline: 1.6885x; 1.6885x over previous
import functools

import jax
import jax.numpy as jnp
from jax import lax
from jax.experimental import pallas as pl
from jax.experimental.pallas import tpu as pltpu
from jax.experimental.pallas import tpu_sc as plsc

N = 50000
E = 800000
F = 78
D = 64
ED = 4
NB = 3
HD = 128

FPAD = 128
EPAD = 8

_NC = 2
_NS = 16
_NW = _NC * _NS
_CHUNK = 128

_EPW = E // _NW
_G_FULL = _EPW // _CHUNK
_G_TAIL = _EPW - _G_FULL * _CHUNK

_EPS = E // _NS
_S_FULL = _EPS // _CHUNK
_S_TAIL = _EPS - _S_FULL * _CHUNK

_NHALF = N // _NC
_ACC = 25024
_RPS = _ACC // _NS
_DUMP = 25016


def _silu(v):
    return v * jax.nn.sigmoid(v)


def _ln(h, g, be):
    mu = jnp.mean(h, axis=-1, keepdims=True)
    var = jnp.mean((h - mu) ** 2, axis=-1, keepdims=True)
    return (h - mu) / jnp.sqrt(var + 1e-5) * g + be


def _dot(a, b):
    return jnp.dot(a, b, preferred_element_type=jnp.float32)


def _wspec(shape):
    nd = len(shape)
    return pl.BlockSpec(shape, lambda i: (0,) * nd)


def _enc_node_body(f_ref, W1, b1, W2, b2, g, be, Wa, Wb,
                   x_ref, ps_ref, pd_ref):
    f = f_ref[...]
    h = _silu(_dot(f, W1[...]) + b1[...])
    x = _ln(_dot(h, W2[...]) + b2[...], g[...], be[...])
    x_ref[...] = x
    ps_ref[...] = _dot(x, Wa[...])
    pd_ref[...] = _dot(x, Wb[...])


def _enc_node_call(fpad, p, Wa, Wb, rows=2000):
    grid = (N // rows,)
    return pl.pallas_call(
        _enc_node_body,
        grid=grid,
        in_specs=[
            pl.BlockSpec((rows, FPAD), lambda i: (i, 0)),
            _wspec((FPAD, D)), _wspec((1, D)), _wspec((D, D)), _wspec((1, D)),
            _wspec((1, D)), _wspec((1, D)), _wspec((D, D)), _wspec((D, D)),
        ],
        out_specs=[
            pl.BlockSpec((rows, D), lambda i: (i, 0)),
            pl.BlockSpec((rows, D), lambda i: (i, 0)),
            pl.BlockSpec((rows, D), lambda i: (i, 0)),
        ],
        out_shape=[
            jax.ShapeDtypeStruct((N, D), jnp.float32),
            jax.ShapeDtypeStruct((N, D), jnp.float32),
            jax.ShapeDtypeStruct((N, D), jnp.float32),
        ],
        compiler_params=pltpu.CompilerParams(
            dimension_semantics=("arbitrary",)),
    )(fpad, p["W1"], p["b1"], p["W2"], p["b2"], p["g"], p["be"], Wa, Wb)


def _edge0_body(ea_ref, gs_ref, gd_ref,
                eW1, eb1, eW2, eb2, eg, ebe,
                W1c, b1, W2, b2, g, be, out_ref):
    ea = ea_ref[...]
    h = _silu(_dot(ea, eW1[...]) + eb1[...])
    e = _ln(_dot(h, eW2[...]) + eb2[...], eg[...], ebe[...])
    pre = gs_ref[...] + gd_ref[...] + _dot(e, W1c[...]) + b1[...]
    hh = _silu(pre)
    m = _ln(_dot(hh, W2[...]) + b2[...], g[...], be[...])
    out_ref[...] = e + m


def _edge0_call(eapad, gs, gd, enc, blk, rows=8000):
    grid = (E // rows,)
    return pl.pallas_call(
        _edge0_body,
        grid=grid,
        in_specs=[
            pl.BlockSpec((rows, EPAD), lambda i: (i, 0)),
            pl.BlockSpec((rows, D), lambda i: (i, 0)),
            pl.BlockSpec((rows, D), lambda i: (i, 0)),
            _wspec((EPAD, D)), _wspec((1, D)), _wspec((D, D)), _wspec((1, D)),
            _wspec((1, D)), _wspec((1, D)),
            _wspec((D, D)), _wspec((1, D)), _wspec((D, D)), _wspec((1, D)),
            _wspec((1, D)), _wspec((1, D)),
        ],
        out_specs=pl.BlockSpec((rows, D), lambda i: (i, 0)),
        out_shape=jax.ShapeDtypeStruct((E, D), jnp.float32),
        compiler_params=pltpu.CompilerParams(
            dimension_semantics=("arbitrary",)),
    )(eapad, gs, gd, enc["W1"], enc["b1"], enc["W2"], enc["b2"], enc["g"],
      enc["be"], blk["W1c"], blk["b1"], blk["W2"], blk["b2"], blk["g"],
      blk["be"])


def _edge_body(e_ref, gs_ref, gd_ref, W1c, b1, W2, b2, g, be, out_ref):
    e = e_ref[...]
    pre = gs_ref[...] + gd_ref[...] + _dot(e, W1c[...]) + b1[...]
    hh = _silu(pre)
    m = _ln(_dot(hh, W2[...]) + b2[...], g[...], be[...])
    out_ref[...] = e + m


def _edge_call(e, gs, gd, blk, rows=8000):
    grid = (E // rows,)
    return pl.pallas_call(
        _edge_body,
        grid=grid,
        in_specs=[
            pl.BlockSpec((rows, D), lambda i: (i, 0)),
            pl.BlockSpec((rows, D), lambda i: (i, 0)),
            pl.BlockSpec((rows, D), lambda i: (i, 0)),
            _wspec((D, D)), _wspec((1, D)), _wspec((D, D)), _wspec((1, D)),
            _wspec((1, D)), _wspec((1, D)),
        ],
        out_specs=pl.BlockSpec((rows, D), lambda i: (i, 0)),
        out_shape=jax.ShapeDtypeStruct((E, D), jnp.float32),
        compiler_params=pltpu.CompilerParams(
            dimension_semantics=("arbitrary",)),
    )(e, gs, gd, blk["W1c"], blk["b1"], blk["W2"], blk["b2"], blk["g"],
      blk["be"])


def _node_body(x_ref, agg_ref, V1a, V1b, b1, W2, b2, g, be, Wa, Wb,
               xo_ref, ps_ref, pd_ref):
    x = x_ref[...]
    pre = _dot(x, V1a[...]) + _dot(agg_ref[...], V1b[...]) + b1[...]
    h = _silu(pre)
    m = _ln(_dot(h, W2[...]) + b2[...], g[...], be[...])
    xn = x + m
    xo_ref[...] = xn
    ps_ref[...] = _dot(xn, Wa[...])
    pd_ref[...] = _dot(xn, Wb[...])


def _node_call(x, agg, blk, Wa, Wb, rows=2000):
    grid = (N // rows,)
    return pl.pallas_call(
        _node_body,
        grid=grid,
        in_specs=[
            pl.BlockSpec((rows, D), lambda i: (i, 0)),
            pl.BlockSpec((rows, D), lambda i: (i, 0)),
            _wspec((D, D)), _wspec((D, D)), _wspec((1, D)), _wspec((D, D)),
            _wspec((1, D)), _wspec((1, D)), _wspec((1, D)),
            _wspec((D, D)), _wspec((D, D)),
        ],
        out_specs=[
            pl.BlockSpec((rows, D), lambda i: (i, 0)),
            pl.BlockSpec((rows, D), lambda i: (i, 0)),
            pl.BlockSpec((rows, D), lambda i: (i, 0)),
        ],
        out_shape=[
            jax.ShapeDtypeStruct((N, D), jnp.float32),
            jax.ShapeDtypeStruct((N, D), jnp.float32),
            jax.ShapeDtypeStruct((N, D), jnp.float32),
        ],
        compiler_params=pltpu.CompilerParams(
            dimension_semantics=("arbitrary",)),
    )(x, agg, blk["V1a"], blk["V1b"], blk["b1"], blk["W2"], blk["b2"],
      blk["g"], blk["be"], Wa, Wb)


def _node_dec_body(x_ref, agg_ref, V1a, V1b, b1, W2, b2, g, be,
                   dW1, db1, dW2, db2, out_ref):
    x = x_ref[...]
    pre = _dot(x, V1a[...]) + _dot(agg_ref[...], V1b[...]) + b1[...]
    h = _silu(pre)
    m = _ln(_dot(h, W2[...]) + b2[...], g[...], be[...])
    xn = x + m
    h2 = _silu(_dot(xn, dW1[...]) + db1[...])
    out_ref[...] = _dot(h2, dW2[...]) + db2[...]


def _node_dec_call(x, agg, blk, dec, rows=2000):
    grid = (N // rows,)
    return pl.pallas_call(
        _node_dec_body,
        grid=grid,
        in_specs=[
            pl.BlockSpec((rows, D), lambda i: (i, 0)),
            pl.BlockSpec((rows, D), lambda i: (i, 0)),
            _wspec((D, D)), _wspec((D, D)), _wspec((1, D)), _wspec((D, D)),
            _wspec((1, D)), _wspec((1, D)), _wspec((1, D)),
            _wspec((D, HD)), _wspec((1, HD)), _wspec((HD, F)), _wspec((1, F)),
        ],
        out_specs=pl.BlockSpec((rows, F), lambda i: (i, 0)),
        out_shape=jax.ShapeDtypeStruct((N, F), jnp.float32),
        compiler_params=pltpu.CompilerParams(
            dimension_semantics=("arbitrary",)),
    )(x, agg, blk["V1a"], blk["V1b"], blk["b1"], blk["W2"], blk["b2"],
      blk["g"], blk["be"], dec["W1"], dec["b1"], dec["W2"], dec["b2"])


@functools.lru_cache(maxsize=None)
def _sc_mesh():
    return plsc.VectorSubcoreMesh(core_axis_name="c", subcore_axis_name="s",
                                  num_cores=_NC, num_subcores=_NS)


def _sc_gather_body(ps_hbm, pd_hbm, src_hbm, dst_hbm, gs_hbm, gd_hbm,
                    idx_s, idx_d, buf_s, buf_d,
                    idx_st, idx_dt, buf_st, buf_dt, sem_s, sem_d):
    wid = lax.axis_index("s") * _NC + lax.axis_index("c")
    eb = wid * _EPW

    def chunk(off, is_, id_, bs, bd, n):
        pltpu.sync_copy(src_hbm.at[pl.ds(off, n)], is_)
        pltpu.sync_copy(dst_hbm.at[pl.ds(off, n)], id_)
        cs = pltpu.async_copy(ps_hbm.at[is_], bs, sem_s)
        cd = pltpu.async_copy(pd_hbm.at[id_], bd, sem_d)
        cs.wait()
        cd.wait()
        pltpu.sync_copy(bs, gs_hbm.at[pl.ds(off, n)])
        pltpu.sync_copy(bd, gd_hbm.at[pl.ds(off, n)])

    def body(i):
        chunk(eb + i * _CHUNK, idx_s, idx_d, buf_s, buf_d, _CHUNK)

    pl.loop(0, _G_FULL)(body)
    chunk(eb + _G_FULL * _CHUNK, idx_st, idx_dt, buf_st, buf_dt, _G_TAIL)


@functools.lru_cache(maxsize=None)
def _sc_gather_kernel():
    @functools.partial(
        pl.kernel,
        mesh=_sc_mesh(),
        out_type=[
            jax.ShapeDtypeStruct((E, D), jnp.float32),
            jax.ShapeDtypeStruct((E, D), jnp.float32),
        ],
        scratch_types=[
            pltpu.VMEM((_CHUNK,), jnp.int32),
            pltpu.VMEM((_CHUNK,), jnp.int32),
            pltpu.VMEM((_CHUNK, D), jnp.float32),
            pltpu.VMEM((_CHUNK, D), jnp.float32),
            pltpu.VMEM((_G_TAIL,), jnp.int32),
            pltpu.VMEM((_G_TAIL,), jnp.int32),
            pltpu.VMEM((_G_TAIL, D), jnp.float32),
            pltpu.VMEM((_G_TAIL, D), jnp.float32),
            pltpu.SemaphoreType.DMA,
            pltpu.SemaphoreType.DMA,
        ],
        compiler_params=pltpu.CompilerParams(use_tc_tiling_on_sc=False),
    )
    def k(ps, pd, src, dst, gs, gd, *scratch):
        _sc_gather_body(ps, pd, src, dst, gs, gd, *scratch)

    return k


def _sc_gather(ps, pd, src, dst):
    return _sc_gather_kernel()(ps, pd, src, dst)


def _sc_scatter_body(en_hbm, dst_hbm, zeros_hbm, out_hbm,
                     acc, rows, idxb, rows_t, idxb_t):
    cid = lax.axis_index("c")
    sid = lax.axis_index("s")
    base = cid * _NHALF
    r0 = sid * _RPS
    pltpu.sync_copy(zeros_hbm.at[pl.ds(r0, _RPS)], acc.at[pl.ds(r0, _RPS)])
    plsc.subcore_barrier()

    def chunk(off, rb, ib, n, kmax):
        pltpu.sync_copy(dst_hbm.at[pl.ds(off, n)], ib)
        pltpu.sync_copy(en_hbm.at[pl.ds(off, n)], rb)
        for k in range(kmax):
            v = ib[pl.ds(k * 16, 16)]
            ok = (v >= base) & (v < base + _NHALF)
            ib[pl.ds(k * 16, 16)] = jnp.where(ok, v - base, _DUMP)
        pltpu.sync_copy(rb, acc.at[ib], add=True)

    sb = sid * _EPS

    def body(i):
        chunk(sb + i * _CHUNK, rows, idxb, _CHUNK, _CHUNK // 16)

    pl.loop(0, _S_FULL)(body)
    chunk(sb + _S_FULL * _CHUNK, rows_t, idxb_t, _S_TAIL, _S_TAIL // 16)

    plsc.subcore_barrier()
    pltpu.sync_copy(acc.at[pl.ds(r0, _RPS)],
                    out_hbm.at[pl.ds(cid * _ACC + r0, _RPS)])


@functools.lru_cache(maxsize=None)
def _sc_scatter_kernel():
    @functools.partial(
        pl.kernel,
        mesh=_sc_mesh(),
        out_type=jax.ShapeDtypeStruct((_NC * _ACC, D), jnp.float32),
        scratch_types=[
            pltpu.VMEM_SHARED((_ACC, D), jnp.float32),
            pltpu.VMEM((_CHUNK, D), jnp.float32),
            pltpu.VMEM((_CHUNK,), jnp.int32),
            pltpu.VMEM((_S_TAIL, D), jnp.float32),
            pltpu.VMEM((_S_TAIL,), jnp.int32),
        ],
        compiler_params=pltpu.CompilerParams(use_tc_tiling_on_sc=False),
    )
    def k(en, dst, zeros, out, *scratch):
        _sc_scatter_body(en, dst, zeros, out, *scratch)

    return k


def _sc_scatter(en, dst, zeros):
    return _sc_scatter_kernel()(en, dst, zeros)


def kernel(features, edge_attr, params, edge_index):
    src = edge_index[0]
    dst = edge_index[1]

    fpad = jnp.pad(features, ((0, 0), (0, FPAD - F)))
    eapad = jnp.pad(edge_attr, ((0, 0), (0, EPAD - ED)))

    enc_node = dict(params["enc_node"])
    enc_node["W1"] = jnp.pad(enc_node["W1"], ((0, FPAD - F), (0, 0)))
    enc_edge = dict(params["enc_edge"])
    enc_edge["W1"] = jnp.pad(enc_edge["W1"], ((0, EPAD - ED), (0, 0)))

    def row(v):
        return v.reshape(1, -1)

    def prep_mlp(p, W1key="W1"):
        return {
            W1key: p["W1"], "b1": row(p["b1"]), "W2": p["W2"],
            "b2": row(p["b2"]), "g": row(p["g"]), "be": row(p["be"]),
        }

    enc_node_p = prep_mlp(enc_node)
    enc_edge_p = prep_mlp(enc_edge)

    eblks = []
    nblks = []
    for blk in params["blocks"]:
        ew = blk["edge"]
        eblks.append({
            "W1a": ew["W1"][0:D], "W1b": ew["W1"][D:2 * D],
            "W1c": ew["W1"][2 * D:3 * D], "b1": row(ew["b1"]),
            "W2": ew["W2"], "b2": row(ew["b2"]),
            "g": row(ew["g"]), "be": row(ew["be"]),
        })
        nw = blk["node"]
        nblks.append({
            "V1a": nw["W1"][0:D], "V1b": nw["W1"][D:2 * D],
            "b1": row(nw["b1"]), "W2": nw["W2"], "b2": row(nw["b2"]),
            "g": row(nw["g"]), "be": row(nw["be"]),
        })

    dec = {
        "W1": params["dec"]["W1"], "b1": row(params["dec"]["b1"]),
        "W2": params["dec"]["W2"], "b2": row(params["dec"]["b2"]),
    }

    zeros = jnp.zeros((_ACC, D), jnp.float32)

    x, ps, pd = _enc_node_call(fpad, enc_node_p, eblks[0]["W1a"],
                               eblks[0]["W1b"])
    e = None
    out = None
    for i in range(NB):
        gs, gd = _sc_gather(ps, pd, src, dst)
        if i == 0:
            e = _edge0_call(eapad, gs, gd, enc_edge_p, eblks[0])
        else:
            e = _edge_call(e, gs, gd, eblks[i])
        aggp = _sc_scatter(e, dst, zeros)
        agg = aggp.reshape(_NC, _ACC, D)[:, :_NHALF].reshape(N, D)
        if i < NB - 1:
            x, ps, pd = _node_call(x, agg, nblks[i], eblks[i + 1]["W1a"],
                                   eblks[i + 1]["W1b"])
        else:
            out = _node_dec_call(x, agg, nblks[i], dec)
    return out


# speedup vs baseline: 2.8839x; 1.7080x over previous
import functools

import jax
import jax.numpy as jnp
from jax import lax
from jax.experimental import pallas as pl
from jax.experimental.pallas import tpu as pltpu
from jax.experimental.pallas import tpu_sc as plsc

N = 50000
E = 800000
F = 78
D = 64
ED = 4
NB = 3
HD = 128
D2 = 2 * D

FPAD = 128
EPAD = 8

_NC = 2
_NS = 16
_NW = _NC * _NS
_CHUNK = 128
_NBUF = 3
_NBUF_S = 2

_EPW = E // _NW
_G_FULL = _EPW // _CHUNK
_G_TAIL = _EPW - _G_FULL * _CHUNK

_EPS = E // _NS
_S_FULL = _EPS // _CHUNK
_S_TAIL = _EPS - _S_FULL * _CHUNK

_NHALF = N // _NC
_ACC = 26000
_RPS = _ACC // _NS
_DUMP = 25600
_NTILE = 1000
_HBLK = _NHALF // _NTILE
_ABLK = _ACC // _NTILE


def _silu(v):
    return v * jax.nn.sigmoid(v)


def _ln(h, g, be):
    mu = jnp.mean(h, axis=-1, keepdims=True)
    var = jnp.mean((h - mu) ** 2, axis=-1, keepdims=True)
    return (h - mu) / jnp.sqrt(var + 1e-5) * g + be


def _dot(a, b):
    return jnp.dot(a, b, preferred_element_type=jnp.float32)


def _wspec(shape):
    nd = len(shape)
    return pl.BlockSpec(shape, lambda i: (0,) * nd)


def _agg_spec():
    return pl.BlockSpec(
        (_NTILE, D2),
        lambda i: (jnp.where(i < _HBLK, i, i + (_ABLK - _HBLK)), 0))


def _enc_node_body(f_ref, W1, b1, W2, b2, g, be, Wa, Wb,
                   x_ref, ps_ref, pd_ref):
    f = f_ref[...]
    h = _silu(_dot(f, W1[...]) + b1[...])
    x = _ln(_dot(h, W2[...]) + b2[...], g[...], be[...])
    x_ref[...] = x
    ps_ref[...] = _dot(x, Wa[...])
    pd_ref[...] = _dot(x, Wb[...])


def _enc_node_call(fpad, p, Wa, Wb):
    grid = (N // _NTILE,)
    return pl.pallas_call(
        _enc_node_body,
        grid=grid,
        in_specs=[
            pl.BlockSpec((_NTILE, FPAD), lambda i: (i, 0)),
            _wspec((FPAD, D)), _wspec((1, D)), _wspec((D, D)), _wspec((1, D)),
            _wspec((1, D)), _wspec((1, D)), _wspec((D, D)), _wspec((D, D)),
        ],
        out_specs=[
            pl.BlockSpec((_NTILE, D), lambda i: (i, 0)),
            pl.BlockSpec((_NTILE, D), lambda i: (i, 0)),
            pl.BlockSpec((_NTILE, D), lambda i: (i, 0)),
        ],
        out_shape=[
            jax.ShapeDtypeStruct((N, D), jnp.float32),
            jax.ShapeDtypeStruct((N, D), jnp.float32),
            jax.ShapeDtypeStruct((N, D), jnp.float32),
        ],
        compiler_params=pltpu.CompilerParams(
            dimension_semantics=("arbitrary",)),
    )(fpad, p["W1"], p["b1"], p["W2"], p["b2"], p["g"], p["be"], Wa, Wb)


def _edge0_body(ea_ref, g_ref,
                eW1, eb1, eW2, eb2, eg, ebe,
                W1c, b1, W2, b2, g, be, out_ref):
    ea = ea_ref[...]
    h = _silu(_dot(ea, eW1[...]) + eb1[...])
    e = _ln(_dot(h, eW2[...]) + eb2[...], eg[...], ebe[...])
    gg = g_ref[...]
    pre = gg[:, :D] + gg[:, D:] + _dot(e, W1c[...]) + b1[...]
    hh = _silu(pre)
    m = _ln(_dot(hh, W2[...]) + b2[...], g[...], be[...])
    en = e + m
    out_ref[...] = jnp.concatenate([en, en], axis=-1)


def _edge0_call(eapad, gath, enc, blk, rows=8000):
    grid = (E // rows,)
    return pl.pallas_call(
        _edge0_body,
        grid=grid,
        in_specs=[
            pl.BlockSpec((rows, EPAD), lambda i: (i, 0)),
            pl.BlockSpec((rows, D2), lambda i: (i, 0)),
            _wspec((EPAD, D)), _wspec((1, D)), _wspec((D, D)), _wspec((1, D)),
            _wspec((1, D)), _wspec((1, D)),
            _wspec((D, D)), _wspec((1, D)), _wspec((D, D)), _wspec((1, D)),
            _wspec((1, D)), _wspec((1, D)),
        ],
        out_specs=pl.BlockSpec((rows, D2), lambda i: (i, 0)),
        out_shape=jax.ShapeDtypeStruct((E, D2), jnp.float32),
        compiler_params=pltpu.CompilerParams(
            dimension_semantics=("arbitrary",)),
    )(eapad, gath, enc["W1"], enc["b1"], enc["W2"], enc["b2"], enc["g"],
      enc["be"], blk["W1c"], blk["b1"], blk["W2"], blk["b2"], blk["g"],
      blk["be"])


def _edge_body(e_ref, g_ref, W1c, b1, W2, b2, g, be, out_ref):
    e = e_ref[...][:, :D]
    gg = g_ref[...]
    pre = gg[:, :D] + gg[:, D:] + _dot(e, W1c[...]) + b1[...]
    hh = _silu(pre)
    m = _ln(_dot(hh, W2[...]) + b2[...], g[...], be[...])
    en = e + m
    out_ref[...] = jnp.concatenate([en, en], axis=-1)


def _edge_call(e2, gath, blk, rows=8000):
    grid = (E // rows,)
    return pl.pallas_call(
        _edge_body,
        grid=grid,
        in_specs=[
            pl.BlockSpec((rows, D2), lambda i: (i, 0)),
            pl.BlockSpec((rows, D2), lambda i: (i, 0)),
            _wspec((D, D)), _wspec((1, D)), _wspec((D, D)), _wspec((1, D)),
            _wspec((1, D)), _wspec((1, D)),
        ],
        out_specs=pl.BlockSpec((rows, D2), lambda i: (i, 0)),
        out_shape=jax.ShapeDtypeStruct((E, D2), jnp.float32),
        compiler_params=pltpu.CompilerParams(
            dimension_semantics=("arbitrary",)),
    )(e2, gath, blk["W1c"], blk["b1"], blk["W2"], blk["b2"], blk["g"],
      blk["be"])


def _node_body(x_ref, agg_ref, V1a, V1b, b1, W2, b2, g, be, Wa, Wb,
               xo_ref, ps_ref, pd_ref):
    x = x_ref[...]
    agg = agg_ref[...][:, :D]
    pre = _dot(x, V1a[...]) + _dot(agg, V1b[...]) + b1[...]
    h = _silu(pre)
    m = _ln(_dot(h, W2[...]) + b2[...], g[...], be[...])
    xn = x + m
    xo_ref[...] = xn
    ps_ref[...] = _dot(xn, Wa[...])
    pd_ref[...] = _dot(xn, Wb[...])


def _node_call(x, aggp, blk, Wa, Wb):
    grid = (N // _NTILE,)
    return pl.pallas_call(
        _node_body,
        grid=grid,
        in_specs=[
            pl.BlockSpec((_NTILE, D), lambda i: (i, 0)),
            _agg_spec(),
            _wspec((D, D)), _wspec((D, D)), _wspec((1, D)), _wspec((D, D)),
            _wspec((1, D)), _wspec((1, D)), _wspec((1, D)),
            _wspec((D, D)), _wspec((D, D)),
        ],
        out_specs=[
            pl.BlockSpec((_NTILE, D), lambda i: (i, 0)),
            pl.BlockSpec((_NTILE, D), lambda i: (i, 0)),
            pl.BlockSpec((_NTILE, D), lambda i: (i, 0)),
        ],
        out_shape=[
            jax.ShapeDtypeStruct((N, D), jnp.float32),
            jax.ShapeDtypeStruct((N, D), jnp.float32),
            jax.ShapeDtypeStruct((N, D), jnp.float32),
        ],
        compiler_params=pltpu.CompilerParams(
            dimension_semantics=("arbitrary",)),
    )(x, aggp, blk["V1a"], blk["V1b"], blk["b1"], blk["W2"], blk["b2"],
      blk["g"], blk["be"], Wa, Wb)


def _node_dec_body(x_ref, agg_ref, V1a, V1b, b1, W2, b2, g, be,
                   dW1, db1, dW2, db2, out_ref):
    x = x_ref[...]
    agg = agg_ref[...][:, :D]
    pre = _dot(x, V1a[...]) + _dot(agg, V1b[...]) + b1[...]
    h = _silu(pre)
    m = _ln(_dot(h, W2[...]) + b2[...], g[...], be[...])
    xn = x + m
    h2 = _silu(_dot(xn, dW1[...]) + db1[...])
    out_ref[...] = _dot(h2, dW2[...]) + db2[...]


def _node_dec_call(x, aggp, blk, dec):
    grid = (N // _NTILE,)
    return pl.pallas_call(
        _node_dec_body,
        grid=grid,
        in_specs=[
            pl.BlockSpec((_NTILE, D), lambda i: (i, 0)),
            _agg_spec(),
            _wspec((D, D)), _wspec((D, D)), _wspec((1, D)), _wspec((D, D)),
            _wspec((1, D)), _wspec((1, D)), _wspec((1, D)),
            _wspec((D, HD)), _wspec((1, HD)), _wspec((HD, F)), _wspec((1, F)),
        ],
        out_specs=pl.BlockSpec((_NTILE, F), lambda i: (i, 0)),
        out_shape=jax.ShapeDtypeStruct((N, F), jnp.float32),
        compiler_params=pltpu.CompilerParams(
            dimension_semantics=("arbitrary",)),
    )(x, aggp, blk["V1a"], blk["V1b"], blk["b1"], blk["W2"], blk["b2"],
      blk["g"], blk["be"], dec["W1"], dec["b1"], dec["W2"], dec["b2"])


@functools.lru_cache(maxsize=None)
def _sc_mesh():
    return plsc.VectorSubcoreMesh(core_axis_name="c", subcore_axis_name="s",
                                  num_cores=_NC, num_subcores=_NS)


def _sc_gather_body(ps_hbm, pd_hbm, src_hbm, dst_hbm, g_hbm,
                    ia, id_, bs, bd, sg, sw,
                    iat, idt, bst, bdt, sgt, swt):
    wid = lax.axis_index("s") * _NC + lax.axis_index("c")
    eb = wid * _EPW

    pltpu.sync_copy(src_hbm.at[pl.ds(eb, _EPW)], ia)
    pltpu.sync_copy(dst_hbm.at[pl.ds(eb, _EPW)], id_)

    def fire(i, b):
        lo = i * _CHUNK
        pltpu.async_copy(ps_hbm.at[ia.at[pl.ds(lo, _CHUNK)]], bs[b], sg[b])
        pltpu.async_copy(pd_hbm.at[id_.at[pl.ds(lo, _CHUNK)]], bd[b], sg[b])

    def drain_gather(b):
        pltpu.make_async_copy(ps_hbm.at[ia.at[pl.ds(0, _CHUNK)]],
                              bs[b], sg[b]).wait()
        pltpu.make_async_copy(pd_hbm.at[id_.at[pl.ds(0, _CHUNK)]],
                              bd[b], sg[b]).wait()

    def fire_write(i, b):
        off = eb + i * _CHUNK
        pltpu.async_copy(bs[b], g_hbm.at[pl.ds(off, _CHUNK), pl.ds(0, D)],
                         sw[b])
        pltpu.async_copy(bd[b], g_hbm.at[pl.ds(off, _CHUNK), pl.ds(D, D)],
                         sw[b])

    def drain_write(b):
        pltpu.make_async_copy(bs[b], g_hbm.at[pl.ds(0, _CHUNK), pl.ds(0, D)],
                              sw[b]).wait()
        pltpu.make_async_copy(bd[b], g_hbm.at[pl.ds(0, _CHUNK), pl.ds(D, D)],
                              sw[b]).wait()

    for b in range(_NBUF):
        fire(b, b)

    def body(i0):
        for b in range(_NBUF):
            i = i0 + b
            drain_gather(b)
            fire_write(i, b)
            drain_write(b)
            fire(i + _NBUF, b)

    pl.loop(0, _G_FULL - _NBUF, step=_NBUF)(body)

    for b in range(_NBUF):
        i = _G_FULL - _NBUF + b
        drain_gather(b)
        fire_write(i, b)
        drain_write(b)

    lo = _G_FULL * _CHUNK
    off = eb + lo
    pltpu.sync_copy(src_hbm.at[pl.ds(off, _G_TAIL)], iat)
    pltpu.sync_copy(dst_hbm.at[pl.ds(off, _G_TAIL)], idt)
    cs = pltpu.async_copy(ps_hbm.at[iat], bst, sgt)
    cd = pltpu.async_copy(pd_hbm.at[idt], bdt, swt)
    cs.wait()
    cd.wait()
    pltpu.sync_copy(bst, g_hbm.at[pl.ds(off, _G_TAIL), pl.ds(0, D)])
    pltpu.sync_copy(bdt, g_hbm.at[pl.ds(off, _G_TAIL), pl.ds(D, D)])


@functools.lru_cache(maxsize=None)
def _sc_gather_kernel():
    @functools.partial(
        pl.kernel,
        mesh=_sc_mesh(),
        out_type=jax.ShapeDtypeStruct((E, D2), jnp.float32),
        scratch_types=[
            pltpu.VMEM((_EPW,), jnp.int32),
            pltpu.VMEM((_EPW,), jnp.int32),
            [pltpu.VMEM((_CHUNK, D), jnp.float32) for _ in range(_NBUF)],
            [pltpu.VMEM((_CHUNK, D), jnp.float32) for _ in range(_NBUF)],
            [pltpu.SemaphoreType.DMA for _ in range(_NBUF)],
            [pltpu.SemaphoreType.DMA for _ in range(_NBUF)],
            pltpu.VMEM((_G_TAIL,), jnp.int32),
            pltpu.VMEM((_G_TAIL,), jnp.int32),
            pltpu.VMEM((_G_TAIL, D), jnp.float32),
            pltpu.VMEM((_G_TAIL, D), jnp.float32),
            pltpu.SemaphoreType.DMA,
            pltpu.SemaphoreType.DMA,
        ],
        compiler_params=pltpu.CompilerParams(use_tc_tiling_on_sc=False),
    )
    def k(ps, pd, src, dst, g, *scratch):
        _sc_gather_body(ps, pd, src, dst, g, *scratch)

    return k


def _sc_gather(ps, pd, src, dst):
    return _sc_gather_kernel()(ps, pd, src, dst)


def _sc_scatter_body(e2_hbm, dst_hbm, zeros_hbm, out_hbm,
                     acc, rbs, ibs, si, sd, ss, rbt, ibt, sit, sdt):
    cid = lax.axis_index("c")
    sid = lax.axis_index("s")
    base = cid * _NHALF
    r0 = sid * _RPS
    pltpu.sync_copy(zeros_hbm.at[pl.ds(r0, _RPS)], acc.at[pl.ds(r0, _RPS)])
    plsc.subcore_barrier()

    sb = sid * _EPS

    def fire(i, b):
        off = sb + i * _CHUNK
        pltpu.async_copy(dst_hbm.at[pl.ds(off, _CHUNK)], ibs[b], si[b])
        pltpu.async_copy(e2_hbm.at[pl.ds(off, _CHUNK), pl.ds(0, D)],
                         rbs[b], sd[b])

    def remap(ib, kmax):
        for k in range(kmax):
            v = ib[pl.ds(k * 16, 16)]
            ok = (v >= base) & (v < base + _NHALF)
            ib[pl.ds(k * 16, 16)] = jnp.where(ok, v - base, _DUMP)

    def process(b):
        pltpu.make_async_copy(dst_hbm.at[pl.ds(0, _CHUNK)], ibs[b],
                              si[b]).wait()
        remap(ibs[b], _CHUNK // 16)
        pltpu.make_async_copy(e2_hbm.at[pl.ds(0, _CHUNK), pl.ds(0, D)],
                              rbs[b], sd[b]).wait()
        pltpu.async_copy(rbs[b], acc.at[ibs[b]], ss[b], add=True)

    def drain_scatter(b):
        pltpu.make_async_copy(rbs[b], acc.at[ibs[b]], ss[b]).wait()

    for b in range(_NBUF_S):
        fire(b, b)

    def body(i0):
        for b in range(_NBUF_S):
            i = i0 + b
            process(b)
            drain_scatter(b)
            fire(i + _NBUF_S, b)

    pl.loop(0, _S_FULL - _NBUF_S, step=_NBUF_S)(body)

    for b in range(_NBUF_S):
        process(b)
        drain_scatter(b)

    off = sb + _S_FULL * _CHUNK
    pltpu.sync_copy(dst_hbm.at[pl.ds(off, _S_TAIL)], ibt)
    remap(ibt, _S_TAIL // 16)
    pltpu.sync_copy(e2_hbm.at[pl.ds(off, _S_TAIL), pl.ds(0, D)], rbt)
    pltpu.sync_copy(rbt, acc.at[ibt], add=True)

    plsc.subcore_barrier()
    pltpu.sync_copy(acc.at[pl.ds(r0, _RPS)],
                    out_hbm.at[pl.ds(cid * _ACC + r0, _RPS), pl.ds(0, D)])


@functools.lru_cache(maxsize=None)
def _sc_scatter_kernel():
    @functools.partial(
        pl.kernel,
        mesh=_sc_mesh(),
        out_type=jax.ShapeDtypeStruct((_NC * _ACC, D2), jnp.float32),
        scratch_types=[
            pltpu.VMEM_SHARED((_ACC, D), jnp.float32),
            [pltpu.VMEM((_CHUNK, D), jnp.float32) for _ in range(_NBUF_S)],
            [pltpu.VMEM((_CHUNK,), jnp.int32) for _ in range(_NBUF_S)],
            [pltpu.SemaphoreType.DMA for _ in range(_NBUF_S)],
            [pltpu.SemaphoreType.DMA for _ in range(_NBUF_S)],
            [pltpu.SemaphoreType.DMA for _ in range(_NBUF_S)],
            pltpu.VMEM((_S_TAIL, D), jnp.float32),
            pltpu.VMEM((_S_TAIL,), jnp.int32),
            pltpu.SemaphoreType.DMA,
            pltpu.SemaphoreType.DMA,
        ],
        compiler_params=pltpu.CompilerParams(use_tc_tiling_on_sc=False),
    )
    def k(e2, dst, zeros, out, *scratch):
        _sc_scatter_body(e2, dst, zeros, out, *scratch)

    return k


def _sc_scatter(e2, dst, zeros):
    return _sc_scatter_kernel()(e2, dst, zeros)


def kernel(features, edge_attr, params, edge_index):
    src = edge_index[0]
    dst = edge_index[1]

    fpad = jnp.pad(features, ((0, 0), (0, FPAD - F)))
    eapad = jnp.pad(edge_attr, ((0, 0), (0, EPAD - ED)))

    enc_node = dict(params["enc_node"])
    enc_node["W1"] = jnp.pad(enc_node["W1"], ((0, FPAD - F), (0, 0)))
    enc_edge = dict(params["enc_edge"])
    enc_edge["W1"] = jnp.pad(enc_edge["W1"], ((0, EPAD - ED), (0, 0)))

    def row(v):
        return v.reshape(1, -1)

    def prep_mlp(p):
        return {
            "W1": p["W1"], "b1": row(p["b1"]), "W2": p["W2"],
            "b2": row(p["b2"]), "g": row(p["g"]), "be": row(p["be"]),
        }

    enc_node_p = prep_mlp(enc_node)
    enc_edge_p = prep_mlp(enc_edge)

    eblks = []
    nblks = []
    for blk in params["blocks"]:
        ew = blk["edge"]
        eblks.append({
            "W1a": ew["W1"][0:D], "W1b": ew["W1"][D:2 * D],
            "W1c": ew["W1"][2 * D:3 * D], "b1": row(ew["b1"]),
            "W2": ew["W2"], "b2": row(ew["b2"]),
            "g": row(ew["g"]), "be": row(ew["be"]),
        })
        nw = blk["node"]
        nblks.append({
            "V1a": nw["W1"][0:D], "V1b": nw["W1"][D:2 * D],
            "b1": row(nw["b1"]), "W2": nw["W2"], "b2": row(nw["b2"]),
            "g": row(nw["g"]), "be": row(nw["be"]),
        })

    dec = {
        "W1": params["dec"]["W1"], "b1": row(params["dec"]["b1"]),
        "W2": params["dec"]["W2"], "b2": row(params["dec"]["b2"]),
    }

    zeros = jnp.zeros((_ACC, D), jnp.float32)

    x, ps, pd = _enc_node_call(fpad, enc_node_p, eblks[0]["W1a"],
                               eblks[0]["W1b"])
    e2 = None
    out = None
    for i in range(NB):
        gath = _sc_gather(ps, pd, src, dst)
        if i == 0:
            e2 = _edge0_call(eapad, gath, enc_edge_p, eblks[0])
        else:
            e2 = _edge_call(e2, gath, eblks[i])
        aggp = _sc_scatter(e2, dst, zeros)
        if i < NB - 1:
            x, ps, pd = _node_call(x, aggp, nblks[i], eblks[i + 1]["W1a"],
                                   eblks[i + 1]["W1b"])
        else:
            out = _node_dec_call(x, aggp, nblks[i], dec)
    return out


# speedup vs baseline: 3.3468x; 1.1605x over previous
import functools

import jax
import jax.numpy as jnp
from jax import lax
from jax.experimental import pallas as pl
from jax.experimental.pallas import tpu as pltpu
from jax.experimental.pallas import tpu_sc as plsc

N = 50000
E = 800000
F = 78
D = 64
ED = 4
NB = 3
HD = 128
D2 = 2 * D

FPAD = 128
EPAD = 8

_NC = 2
_NS = 16
_NW = _NC * _NS
_CHUNK = 128
_NBUF = 3
_NBUF_S = 2

_EPW = E // _NW
_G_FULL = _EPW // _CHUNK
_G_TAIL = _EPW - _G_FULL * _CHUNK

_EPS = E // _NS
_S_CHUNK = 128
_S_FULL = _EPS // _S_CHUNK
_S_TAIL = _EPS - _S_FULL * _S_CHUNK

_NHALF = N // _NC
_ACC = 26000
_RPS = _ACC // _NS
_DUMP = 25600
_NTILE = 1000
_HBLK = _NHALF // _NTILE
_ABLK = _ACC // _NTILE


def _silu(v):
    return v * jax.nn.sigmoid(v)


def _ln(h, g, be):
    mu = jnp.mean(h, axis=-1, keepdims=True)
    var = jnp.mean((h - mu) ** 2, axis=-1, keepdims=True)
    return (h - mu) / jnp.sqrt(var + 1e-5) * g + be


def _dot(a, b):
    return jnp.dot(a, b, preferred_element_type=jnp.float32)


def _wspec(shape):
    nd = len(shape)
    return pl.BlockSpec(shape, lambda i: (0,) * nd)


def _agg_spec():
    return pl.BlockSpec(
        (_NTILE, D2),
        lambda i: (jnp.where(i < _HBLK, i, i + (_ABLK - _HBLK)), 0))


def _enc_node_body(f_ref, W1, b1, W2, b2, g, be, Wa, Wb,
                   x_ref, ps_ref, pd_ref):
    f = f_ref[...]
    h = _silu(_dot(f, W1[...]) + b1[...])
    x = _ln(_dot(h, W2[...]) + b2[...], g[...], be[...])
    x_ref[...] = x
    ps_ref[...] = _dot(x, Wa[...])
    pd_ref[...] = _dot(x, Wb[...])


def _enc_node_call(fpad, p, Wa, Wb):
    grid = (N // _NTILE,)
    return pl.pallas_call(
        _enc_node_body,
        grid=grid,
        in_specs=[
            pl.BlockSpec((_NTILE, F), lambda i: (i, 0)),
            _wspec((F, D)), _wspec((1, D)), _wspec((D, D)), _wspec((1, D)),
            _wspec((1, D)), _wspec((1, D)), _wspec((D, D)), _wspec((D, D)),
        ],
        out_specs=[
            pl.BlockSpec((_NTILE, D), lambda i: (i, 0)),
            pl.BlockSpec((_NTILE, D), lambda i: (i, 0)),
            pl.BlockSpec((_NTILE, D), lambda i: (i, 0)),
        ],
        out_shape=[
            jax.ShapeDtypeStruct((N, D), jnp.float32),
            jax.ShapeDtypeStruct((N, D), jnp.float32),
            jax.ShapeDtypeStruct((N, D), jnp.float32),
        ],
        compiler_params=pltpu.CompilerParams(
            dimension_semantics=("arbitrary",)),
    )(fpad, p["W1"], p["b1"], p["W2"], p["b2"], p["g"], p["be"], Wa, Wb)


def _edge0_body(ea_ref, g_ref,
                eW1, eb1, eW2, eb2, eg, ebe,
                W1c, b1, W2, b2, g, be, out_ref):
    ea = ea_ref[...]
    h = _silu(_dot(ea, eW1[...]) + eb1[...])
    e = _ln(_dot(h, eW2[...]) + eb2[...], eg[...], ebe[...])
    gg = g_ref[...]
    pre = gg[:, :D] + gg[:, D:] + _dot(e, W1c[...]) + b1[...]
    hh = _silu(pre)
    m = _ln(_dot(hh, W2[...]) + b2[...], g[...], be[...])
    en = e + m
    out_ref[...] = jnp.concatenate([en, en], axis=-1)


def _edge0_call(eapad, gath, enc, blk, rows=8000):
    grid = (E // rows,)
    return pl.pallas_call(
        _edge0_body,
        grid=grid,
        in_specs=[
            pl.BlockSpec((rows, ED), lambda i: (i, 0)),
            pl.BlockSpec((rows, D2), lambda i: (i, 0)),
            _wspec((ED, D)), _wspec((1, D)), _wspec((D, D)), _wspec((1, D)),
            _wspec((1, D)), _wspec((1, D)),
            _wspec((D, D)), _wspec((1, D)), _wspec((D, D)), _wspec((1, D)),
            _wspec((1, D)), _wspec((1, D)),
        ],
        out_specs=pl.BlockSpec((rows, D2), lambda i: (i, 0)),
        out_shape=jax.ShapeDtypeStruct((E, D2), jnp.float32),
        compiler_params=pltpu.CompilerParams(
            dimension_semantics=("arbitrary",)),
    )(eapad, gath, enc["W1"], enc["b1"], enc["W2"], enc["b2"], enc["g"],
      enc["be"], blk["W1c"], blk["b1"], blk["W2"], blk["b2"], blk["g"],
      blk["be"])


def _edge_body(e_ref, g_ref, W1c, b1, W2, b2, g, be, out_ref):
    e = e_ref[...][:, :D]
    gg = g_ref[...]
    pre = gg[:, :D] + gg[:, D:] + _dot(e, W1c[...]) + b1[...]
    hh = _silu(pre)
    m = _ln(_dot(hh, W2[...]) + b2[...], g[...], be[...])
    en = e + m
    out_ref[...] = jnp.concatenate([en, en], axis=-1)


def _edge_call(e2, gath, blk, rows=8000):
    grid = (E // rows,)
    return pl.pallas_call(
        _edge_body,
        grid=grid,
        in_specs=[
            pl.BlockSpec((rows, D2), lambda i: (i, 0)),
            pl.BlockSpec((rows, D2), lambda i: (i, 0)),
            _wspec((D, D)), _wspec((1, D)), _wspec((D, D)), _wspec((1, D)),
            _wspec((1, D)), _wspec((1, D)),
        ],
        out_specs=pl.BlockSpec((rows, D2), lambda i: (i, 0)),
        out_shape=jax.ShapeDtypeStruct((E, D2), jnp.float32),
        compiler_params=pltpu.CompilerParams(
            dimension_semantics=("arbitrary",)),
    )(e2, gath, blk["W1c"], blk["b1"], blk["W2"], blk["b2"], blk["g"],
      blk["be"])


def _node_body(x_ref, agg_ref, V1a, V1b, b1, W2, b2, g, be, Wa, Wb,
               xo_ref, ps_ref, pd_ref):
    x = x_ref[...]
    agg = agg_ref[...][:, :D]
    pre = _dot(x, V1a[...]) + _dot(agg, V1b[...]) + b1[...]
    h = _silu(pre)
    m = _ln(_dot(h, W2[...]) + b2[...], g[...], be[...])
    xn = x + m
    xo_ref[...] = xn
    ps_ref[...] = _dot(xn, Wa[...])
    pd_ref[...] = _dot(xn, Wb[...])


def _node_call(x, aggp, blk, Wa, Wb):
    grid = (N // _NTILE,)
    return pl.pallas_call(
        _node_body,
        grid=grid,
        in_specs=[
            pl.BlockSpec((_NTILE, D), lambda i: (i, 0)),
            _agg_spec(),
            _wspec((D, D)), _wspec((D, D)), _wspec((1, D)), _wspec((D, D)),
            _wspec((1, D)), _wspec((1, D)), _wspec((1, D)),
            _wspec((D, D)), _wspec((D, D)),
        ],
        out_specs=[
            pl.BlockSpec((_NTILE, D), lambda i: (i, 0)),
            pl.BlockSpec((_NTILE, D), lambda i: (i, 0)),
            pl.BlockSpec((_NTILE, D), lambda i: (i, 0)),
        ],
        out_shape=[
            jax.ShapeDtypeStruct((N, D), jnp.float32),
            jax.ShapeDtypeStruct((N, D), jnp.float32),
            jax.ShapeDtypeStruct((N, D), jnp.float32),
        ],
        compiler_params=pltpu.CompilerParams(
            dimension_semantics=("arbitrary",)),
    )(x, aggp, blk["V1a"], blk["V1b"], blk["b1"], blk["W2"], blk["b2"],
      blk["g"], blk["be"], Wa, Wb)


def _node_dec_body(x_ref, agg_ref, V1a, V1b, b1, W2, b2, g, be,
                   dW1, db1, dW2, db2, out_ref):
    x = x_ref[...]
    agg = agg_ref[...][:, :D]
    pre = _dot(x, V1a[...]) + _dot(agg, V1b[...]) + b1[...]
    h = _silu(pre)
    m = _ln(_dot(h, W2[...]) + b2[...], g[...], be[...])
    xn = x + m
    h2 = _silu(_dot(xn, dW1[...]) + db1[...])
    out_ref[...] = _dot(h2, dW2[...]) + db2[...]


def _node_dec_call(x, aggp, blk, dec):
    grid = (N // _NTILE,)
    return pl.pallas_call(
        _node_dec_body,
        grid=grid,
        in_specs=[
            pl.BlockSpec((_NTILE, D), lambda i: (i, 0)),
            _agg_spec(),
            _wspec((D, D)), _wspec((D, D)), _wspec((1, D)), _wspec((D, D)),
            _wspec((1, D)), _wspec((1, D)), _wspec((1, D)),
            _wspec((D, HD)), _wspec((1, HD)), _wspec((HD, F)), _wspec((1, F)),
        ],
        out_specs=pl.BlockSpec((_NTILE, F), lambda i: (i, 0)),
        out_shape=jax.ShapeDtypeStruct((N, F), jnp.float32),
        compiler_params=pltpu.CompilerParams(
            dimension_semantics=("arbitrary",)),
    )(x, aggp, blk["V1a"], blk["V1b"], blk["b1"], blk["W2"], blk["b2"],
      blk["g"], blk["be"], dec["W1"], dec["b1"], dec["W2"], dec["b2"])


@functools.lru_cache(maxsize=None)
def _sc_mesh():
    return plsc.VectorSubcoreMesh(core_axis_name="c", subcore_axis_name="s",
                                  num_cores=_NC, num_subcores=_NS)


def _sc_gather_body(ps_hbm, pd_hbm, src_hbm, dst_hbm, g_hbm,
                    ia, id_, bs, bd, sg, sw,
                    iat, idt, bst, bdt, sgt, swt):
    wid = lax.axis_index("s") * _NC + lax.axis_index("c")
    eb = wid * _EPW

    pltpu.sync_copy(src_hbm.at[pl.ds(eb, _EPW)], ia)
    pltpu.sync_copy(dst_hbm.at[pl.ds(eb, _EPW)], id_)

    def fire(i, b):
        lo = i * _CHUNK
        pltpu.async_copy(ps_hbm.at[ia.at[pl.ds(lo, _CHUNK)]], bs[b], sg[b])
        pltpu.async_copy(pd_hbm.at[id_.at[pl.ds(lo, _CHUNK)]], bd[b], sg[b])

    def drain_gather(b):
        pltpu.make_async_copy(ps_hbm.at[ia.at[pl.ds(0, _CHUNK)]],
                              bs[b], sg[b]).wait()
        pltpu.make_async_copy(pd_hbm.at[id_.at[pl.ds(0, _CHUNK)]],
                              bd[b], sg[b]).wait()

    def fire_write(i, b):
        off = eb + i * _CHUNK
        pltpu.async_copy(bs[b], g_hbm.at[pl.ds(off, _CHUNK), pl.ds(0, D)],
                         sw[b])
        pltpu.async_copy(bd[b], g_hbm.at[pl.ds(off, _CHUNK), pl.ds(D, D)],
                         sw[b])

    def drain_write(b):
        pltpu.make_async_copy(bs[b], g_hbm.at[pl.ds(0, _CHUNK), pl.ds(0, D)],
                              sw[b]).wait()
        pltpu.make_async_copy(bd[b], g_hbm.at[pl.ds(0, _CHUNK), pl.ds(D, D)],
                              sw[b]).wait()

    for b in range(_NBUF):
        fire(b, b)

    def body(i0):
        for b in range(_NBUF):
            i = i0 + b
            drain_gather(b)
            fire_write(i, b)
            drain_write(b)
            fire(i + _NBUF, b)

    pl.loop(0, _G_FULL - _NBUF, step=_NBUF)(body)

    for b in range(_NBUF):
        i = _G_FULL - _NBUF + b
        drain_gather(b)
        fire_write(i, b)
        drain_write(b)

    lo = _G_FULL * _CHUNK
    off = eb + lo
    pltpu.sync_copy(src_hbm.at[pl.ds(off, _G_TAIL)], iat)
    pltpu.sync_copy(dst_hbm.at[pl.ds(off, _G_TAIL)], idt)
    cs = pltpu.async_copy(ps_hbm.at[iat], bst, sgt)
    cd = pltpu.async_copy(pd_hbm.at[idt], bdt, swt)
    cs.wait()
    cd.wait()
    pltpu.sync_copy(bst, g_hbm.at[pl.ds(off, _G_TAIL), pl.ds(0, D)])
    pltpu.sync_copy(bdt, g_hbm.at[pl.ds(off, _G_TAIL), pl.ds(D, D)])


@functools.lru_cache(maxsize=None)
def _sc_gather_kernel():
    @functools.partial(
        pl.kernel,
        mesh=_sc_mesh(),
        out_type=jax.ShapeDtypeStruct((E, D2), jnp.float32),
        scratch_types=[
            pltpu.VMEM((_EPW,), jnp.int32),
            pltpu.VMEM((_EPW,), jnp.int32),
            [pltpu.VMEM((_CHUNK, D), jnp.float32) for _ in range(_NBUF)],
            [pltpu.VMEM((_CHUNK, D), jnp.float32) for _ in range(_NBUF)],
            [pltpu.SemaphoreType.DMA for _ in range(_NBUF)],
            [pltpu.SemaphoreType.DMA for _ in range(_NBUF)],
            pltpu.VMEM((_G_TAIL,), jnp.int32),
            pltpu.VMEM((_G_TAIL,), jnp.int32),
            pltpu.VMEM((_G_TAIL, D), jnp.float32),
            pltpu.VMEM((_G_TAIL, D), jnp.float32),
            pltpu.SemaphoreType.DMA,
            pltpu.SemaphoreType.DMA,
        ],
        compiler_params=pltpu.CompilerParams(use_tc_tiling_on_sc=False),
    )
    def k(ps, pd, src, dst, g, *scratch):
        _sc_gather_body(ps, pd, src, dst, g, *scratch)

    return k


def _sc_gather(ps, pd, src, dst):
    return _sc_gather_kernel()(ps, pd, src, dst)


def _sc_scatter_body(e2_hbm, dst_hbm, zeros_hbm, out_hbm,
                     acc, rbs, ibs, si, sd, ss, ibt):
    cid = lax.axis_index("c")
    sid = lax.axis_index("s")
    base = cid * _NHALF
    r0 = sid * _RPS
    pltpu.sync_copy(zeros_hbm.at[pl.ds(r0, _RPS)], acc.at[pl.ds(r0, _RPS)])
    plsc.subcore_barrier()

    sb = sid * _EPS

    def fire(i, b):
        off = sb + i * _S_CHUNK
        pltpu.async_copy(dst_hbm.at[pl.ds(off, _S_CHUNK)], ibs[b], si[b])
        pltpu.async_copy(e2_hbm.at[pl.ds(off, _S_CHUNK), pl.ds(0, D)],
                         rbs[b], sd[b])

    def remap(ib, kmax):
        for k in range(kmax):
            v = ib[pl.ds(k * 16, 16)]
            ok = (v >= base) & (v < base + _NHALF)
            ib[pl.ds(k * 16, 16)] = jnp.where(ok, v - base, _DUMP)

    def process(b):
        pltpu.make_async_copy(dst_hbm.at[pl.ds(0, _S_CHUNK)], ibs[b],
                              si[b]).wait()
        remap(ibs[b], _S_CHUNK // 16)
        pltpu.make_async_copy(e2_hbm.at[pl.ds(0, _S_CHUNK), pl.ds(0, D)],
                              rbs[b], sd[b]).wait()
        pltpu.async_copy(rbs[b], acc.at[ibs[b]], ss[b], add=True)

    def drain_scatter(b):
        pltpu.make_async_copy(rbs[b], acc.at[ibs[b]], ss[b]).wait()

    for b in range(_NBUF_S):
        fire(b, b)

    def body(i0):
        for b in range(_NBUF_S):
            i = i0 + b
            process(b)
            drain_scatter(b)
            fire(i + _NBUF_S, b)

    pl.loop(0, _S_FULL - _NBUF_S, step=_NBUF_S)(body)

    for b in range(_NBUF_S):
        process(b)
        drain_scatter(b)

    off = sb + _S_FULL * _S_CHUNK
    pltpu.sync_copy(dst_hbm.at[pl.ds(off, _S_TAIL)], ibt)
    remap(ibt, _S_TAIL // 16)
    pltpu.sync_copy(e2_hbm.at[pl.ds(off, _S_TAIL), pl.ds(0, D)],
                    rbs[0].at[pl.ds(0, _S_TAIL)])
    pltpu.sync_copy(rbs[0].at[pl.ds(0, _S_TAIL)], acc.at[ibt], add=True)

    plsc.subcore_barrier()
    pltpu.sync_copy(acc.at[pl.ds(r0, _RPS)],
                    out_hbm.at[pl.ds(cid * _ACC + r0, _RPS), pl.ds(0, D)])


@functools.lru_cache(maxsize=None)
def _sc_scatter_kernel():
    @functools.partial(
        pl.kernel,
        mesh=_sc_mesh(),
        out_type=jax.ShapeDtypeStruct((_NC * _ACC, D2), jnp.float32),
        scratch_types=[
            pltpu.VMEM_SHARED((_ACC, D), jnp.float32),
            [pltpu.VMEM((_S_CHUNK, D), jnp.float32) for _ in range(_NBUF_S)],
            [pltpu.VMEM((_S_CHUNK,), jnp.int32) for _ in range(_NBUF_S)],
            [pltpu.SemaphoreType.DMA for _ in range(_NBUF_S)],
            [pltpu.SemaphoreType.DMA for _ in range(_NBUF_S)],
            [pltpu.SemaphoreType.DMA for _ in range(_NBUF_S)],
            pltpu.VMEM((_S_TAIL,), jnp.int32),
        ],
        compiler_params=pltpu.CompilerParams(use_tc_tiling_on_sc=False),
    )
    def k(e2, dst, zeros, out, *scratch):
        _sc_scatter_body(e2, dst, zeros, out, *scratch)

    return k


def _sc_scatter(e2, dst, zeros):
    return _sc_scatter_kernel()(e2, dst, zeros)


def kernel(features, edge_attr, params, edge_index):
    src = edge_index[0]
    dst = edge_index[1]

    fpad = features
    eapad = edge_attr

    enc_node = dict(params["enc_node"])
    enc_edge = dict(params["enc_edge"])

    def row(v):
        return v.reshape(1, -1)

    def prep_mlp(p):
        return {
            "W1": p["W1"], "b1": row(p["b1"]), "W2": p["W2"],
            "b2": row(p["b2"]), "g": row(p["g"]), "be": row(p["be"]),
        }

    enc_node_p = prep_mlp(enc_node)
    enc_edge_p = prep_mlp(enc_edge)

    eblks = []
    nblks = []
    for blk in params["blocks"]:
        ew = blk["edge"]
        eblks.append({
            "W1a": ew["W1"][0:D], "W1b": ew["W1"][D:2 * D],
            "W1c": ew["W1"][2 * D:3 * D], "b1": row(ew["b1"]),
            "W2": ew["W2"], "b2": row(ew["b2"]),
            "g": row(ew["g"]), "be": row(ew["be"]),
        })
        nw = blk["node"]
        nblks.append({
            "V1a": nw["W1"][0:D], "V1b": nw["W1"][D:2 * D],
            "b1": row(nw["b1"]), "W2": nw["W2"], "b2": row(nw["b2"]),
            "g": row(nw["g"]), "be": row(nw["be"]),
        })

    dec = {
        "W1": params["dec"]["W1"], "b1": row(params["dec"]["b1"]),
        "W2": params["dec"]["W2"], "b2": row(params["dec"]["b2"]),
    }

    zeros = jnp.zeros((_ACC, D), jnp.float32)

    x, ps, pd = _enc_node_call(fpad, enc_node_p, eblks[0]["W1a"],
                               eblks[0]["W1b"])
    e2 = None
    out = None
    for i in range(NB):
        gath = _sc_gather(ps, pd, src, dst)
        if i == 0:
            e2 = _edge0_call(eapad, gath, enc_edge_p, eblks[0])
        else:
            e2 = _edge_call(e2, gath, eblks[i])
        aggp = _sc_scatter(e2, dst, zeros)
        if i < NB - 1:
            x, ps, pd = _node_call(x, aggp, nblks[i], eblks[i + 1]["W1a"],
                                   eblks[i + 1]["W1b"])
        else:
            out = _node_dec_call(x, aggp, nblks[i], dec)
    return out


# speedup vs baseline: 3.9145x; 1.1696x over previous
import functools

import jax
import jax.numpy as jnp
from jax import lax
from jax.experimental import pallas as pl
from jax.experimental.pallas import tpu as pltpu
from jax.experimental.pallas import tpu_sc as plsc

N = 50000
E = 800000
F = 78
D = 64
ED = 4
NB = 3
HD = 128
D2 = 2 * D

FPAD = 128
EPAD = 8

_NC = 2
_NS = 16
_NW = _NC * _NS
_CHUNK = 128
_NBUF = 2
_NBUF_S = 2

_EHA = 409600
_EHB = E - _EHA
_EOFF = (0, _EHA)
_EHALF = (_EHA, _EHB)

_EPW = tuple(h // _NW for h in _EHALF)
_G_FULL = tuple(w // _CHUNK for w in _EPW)
_G_TAIL = tuple(w - f * _CHUNK for w, f in zip(_EPW, _G_FULL))

_EPS = tuple(h // _NS for h in _EHALF)
_S_CHUNK = 128
_S_FULL = tuple(p // _S_CHUNK for p in _EPS)
_S_TAIL = tuple(p - f * _S_CHUNK for p, f in zip(_EPS, _S_FULL))

_NHALF = N // _NC
_ACC = 26000
_RPS = _ACC // _NS
_DUMP = 25600
_NTILE = 1000
_HBLK = _NHALF // _NTILE
_ABLK = _ACC // _NTILE


def _silu(v):
    return v * jax.nn.sigmoid(v)


def _ln(h, g, be):
    mu = jnp.mean(h, axis=-1, keepdims=True)
    var = jnp.mean((h - mu) ** 2, axis=-1, keepdims=True)
    return (h - mu) / jnp.sqrt(var + 1e-5) * g + be


def _dot(a, b):
    return jnp.dot(a, b, preferred_element_type=jnp.float32)


def _wspec(shape):
    nd = len(shape)
    return pl.BlockSpec(shape, lambda i: (0,) * nd)


def _agg_spec():
    return pl.BlockSpec(
        (_NTILE, D2),
        lambda i: (jnp.where(i < _HBLK, i, i + (_ABLK - _HBLK)), 0))


def _enc_node_body(f_ref, W1, b1, W2, b2, g, be, Wa, Wb,
                   x_ref, ps_ref, pd_ref):
    f = f_ref[...]
    h = _silu(_dot(f, W1[...]) + b1[...])
    x = _ln(_dot(h, W2[...]) + b2[...], g[...], be[...])
    x_ref[...] = x
    ps_ref[...] = _dot(x, Wa[...])
    pd_ref[...] = _dot(x, Wb[...])


def _enc_node_call(fpad, p, Wa, Wb):
    grid = (N // _NTILE,)
    return pl.pallas_call(
        _enc_node_body,
        grid=grid,
        in_specs=[
            pl.BlockSpec((_NTILE, F), lambda i: (i, 0)),
            _wspec((F, D)), _wspec((1, D)), _wspec((D, D)), _wspec((1, D)),
            _wspec((1, D)), _wspec((1, D)), _wspec((D, D)), _wspec((D, D)),
        ],
        out_specs=[
            pl.BlockSpec((_NTILE, D), lambda i: (i, 0)),
            pl.BlockSpec((_NTILE, D), lambda i: (i, 0)),
            pl.BlockSpec((_NTILE, D), lambda i: (i, 0)),
        ],
        out_shape=[
            jax.ShapeDtypeStruct((N, D), jnp.float32),
            jax.ShapeDtypeStruct((N, D), jnp.float32),
            jax.ShapeDtypeStruct((N, D), jnp.float32),
        ],
        compiler_params=pltpu.CompilerParams(
            dimension_semantics=("arbitrary",)),
    )(fpad, p["W1"], p["b1"], p["W2"], p["b2"], p["g"], p["be"], Wa, Wb)


def _edge0_body(ea_ref, g_ref,
                eW1, eb1, eW2, eb2, eg, ebe,
                W1c, b1, W2, b2, g, be, out_ref):
    ea = ea_ref[...]
    h = _silu(_dot(ea, eW1[...]) + eb1[...])
    e = _ln(_dot(h, eW2[...]) + eb2[...], eg[...], ebe[...])
    gg = g_ref[...]
    pre = gg[:, :D] + gg[:, D:] + _dot(e, W1c[...]) + b1[...]
    hh = _silu(pre)
    m = _ln(_dot(hh, W2[...]) + b2[...], g[...], be[...])
    en = e + m
    out_ref[...] = jnp.concatenate([en, en], axis=-1)


def _edge0_call(eapad, gath, enc, blk, half, rows=6400):
    grid = (_EHALF[half] // rows,)
    boff = _EOFF[half] // rows
    return pl.pallas_call(
        _edge0_body,
        grid=grid,
        in_specs=[
            pl.BlockSpec((rows, ED), lambda i: (i + boff, 0)),
            pl.BlockSpec((rows, D2), lambda i: (i, 0)),
            _wspec((ED, D)), _wspec((1, D)), _wspec((D, D)), _wspec((1, D)),
            _wspec((1, D)), _wspec((1, D)),
            _wspec((D, D)), _wspec((1, D)), _wspec((D, D)), _wspec((1, D)),
            _wspec((1, D)), _wspec((1, D)),
        ],
        out_specs=pl.BlockSpec((rows, D2), lambda i: (i, 0)),
        out_shape=jax.ShapeDtypeStruct((_EHALF[half], D2), jnp.float32),
        compiler_params=pltpu.CompilerParams(
            dimension_semantics=("arbitrary",)),
    )(eapad, gath, enc["W1"], enc["b1"], enc["W2"], enc["b2"], enc["g"],
      enc["be"], blk["W1c"], blk["b1"], blk["W2"], blk["b2"], blk["g"],
      blk["be"])


def _edge_body(e_ref, g_ref, W1c, b1, W2, b2, g, be, out_ref):
    e = e_ref[...][:, :D]
    gg = g_ref[...]
    pre = gg[:, :D] + gg[:, D:] + _dot(e, W1c[...]) + b1[...]
    hh = _silu(pre)
    m = _ln(_dot(hh, W2[...]) + b2[...], g[...], be[...])
    en = e + m
    out_ref[...] = jnp.concatenate([en, en], axis=-1)


def _edge_call(e2, gath, blk, rows=6400):
    grid = (e2.shape[0] // rows,)
    return pl.pallas_call(
        _edge_body,
        grid=grid,
        in_specs=[
            pl.BlockSpec((rows, D2), lambda i: (i, 0)),
            pl.BlockSpec((rows, D2), lambda i: (i, 0)),
            _wspec((D, D)), _wspec((1, D)), _wspec((D, D)), _wspec((1, D)),
            _wspec((1, D)), _wspec((1, D)),
        ],
        out_specs=pl.BlockSpec((rows, D2), lambda i: (i, 0)),
        out_shape=jax.ShapeDtypeStruct(e2.shape, jnp.float32),
        compiler_params=pltpu.CompilerParams(
            dimension_semantics=("arbitrary",)),
    )(e2, gath, blk["W1c"], blk["b1"], blk["W2"], blk["b2"], blk["g"],
      blk["be"])


def _node_body(x_ref, aggA_ref, aggB_ref, V1a, V1b, b1, W2, b2, g, be,
               Wa, Wb, xo_ref, ps_ref, pd_ref):
    x = x_ref[...]
    agg = aggA_ref[...][:, :D] + aggB_ref[...][:, :D]
    pre = _dot(x, V1a[...]) + _dot(agg, V1b[...]) + b1[...]
    h = _silu(pre)
    m = _ln(_dot(h, W2[...]) + b2[...], g[...], be[...])
    xn = x + m
    xo_ref[...] = xn
    ps_ref[...] = _dot(xn, Wa[...])
    pd_ref[...] = _dot(xn, Wb[...])


def _node_call(x, aggA, aggB, blk, Wa, Wb):
    grid = (N // _NTILE,)
    return pl.pallas_call(
        _node_body,
        grid=grid,
        in_specs=[
            pl.BlockSpec((_NTILE, D), lambda i: (i, 0)),
            _agg_spec(),
            _agg_spec(),
            _wspec((D, D)), _wspec((D, D)), _wspec((1, D)), _wspec((D, D)),
            _wspec((1, D)), _wspec((1, D)), _wspec((1, D)),
            _wspec((D, D)), _wspec((D, D)),
        ],
        out_specs=[
            pl.BlockSpec((_NTILE, D), lambda i: (i, 0)),
            pl.BlockSpec((_NTILE, D), lambda i: (i, 0)),
            pl.BlockSpec((_NTILE, D), lambda i: (i, 0)),
        ],
        out_shape=[
            jax.ShapeDtypeStruct((N, D), jnp.float32),
            jax.ShapeDtypeStruct((N, D), jnp.float32),
            jax.ShapeDtypeStruct((N, D), jnp.float32),
        ],
        compiler_params=pltpu.CompilerParams(
            dimension_semantics=("arbitrary",)),
    )(x, aggA, aggB, blk["V1a"], blk["V1b"], blk["b1"], blk["W2"],
      blk["b2"], blk["g"], blk["be"], Wa, Wb)


def _node_dec_body(x_ref, aggA_ref, aggB_ref, V1a, V1b, b1, W2, b2, g, be,
                   dW1, db1, dW2, db2, out_ref):
    x = x_ref[...]
    agg = aggA_ref[...][:, :D] + aggB_ref[...][:, :D]
    pre = _dot(x, V1a[...]) + _dot(agg, V1b[...]) + b1[...]
    h = _silu(pre)
    m = _ln(_dot(h, W2[...]) + b2[...], g[...], be[...])
    xn = x + m
    h2 = _silu(_dot(xn, dW1[...]) + db1[...])
    out_ref[...] = _dot(h2, dW2[...]) + db2[...]


def _node_dec_call(x, aggA, aggB, blk, dec):
    grid = (N // _NTILE,)
    return pl.pallas_call(
        _node_dec_body,
        grid=grid,
        in_specs=[
            pl.BlockSpec((_NTILE, D), lambda i: (i, 0)),
            _agg_spec(),
            _agg_spec(),
            _wspec((D, D)), _wspec((D, D)), _wspec((1, D)), _wspec((D, D)),
            _wspec((1, D)), _wspec((1, D)), _wspec((1, D)),
            _wspec((D, HD)), _wspec((1, HD)), _wspec((HD, F)), _wspec((1, F)),
        ],
        out_specs=pl.BlockSpec((_NTILE, F), lambda i: (i, 0)),
        out_shape=jax.ShapeDtypeStruct((N, F), jnp.float32),
        compiler_params=pltpu.CompilerParams(
            dimension_semantics=("arbitrary",)),
    )(x, aggA, aggB, blk["V1a"], blk["V1b"], blk["b1"], blk["W2"],
      blk["b2"], blk["g"], blk["be"], dec["W1"], dec["b1"], dec["W2"],
      dec["b2"])


@functools.lru_cache(maxsize=None)
def _sc_mesh():
    return plsc.VectorSubcoreMesh(core_axis_name="c", subcore_axis_name="s",
                                  num_cores=_NC, num_subcores=_NS)


def _sc_gather_body(half, ps_hbm, pd_hbm, src_hbm, dst_hbm, g_hbm,
                    ia, id_, bs, bd, sg, sw,
                    iat, idt, bst, bdt, sgt, swt):
    epw = _EPW[half]
    full = _G_FULL[half]
    tail = _G_TAIL[half]
    ring = full - full % _NBUF

    wid = lax.axis_index("s") * _NC + lax.axis_index("c")
    lb = wid * epw
    eb = _EOFF[half] + lb

    pltpu.sync_copy(src_hbm.at[pl.ds(eb, epw)], ia)
    pltpu.sync_copy(dst_hbm.at[pl.ds(eb, epw)], id_)

    def fire(i, b):
        lo = i * _CHUNK
        pltpu.async_copy(ps_hbm.at[ia.at[pl.ds(lo, _CHUNK)]], bs[b], sg[b])
        pltpu.async_copy(pd_hbm.at[id_.at[pl.ds(lo, _CHUNK)]], bd[b], sg[b])

    def drain_gather(b):
        pltpu.make_async_copy(ps_hbm.at[ia.at[pl.ds(0, _CHUNK)]],
                              bs[b], sg[b]).wait()
        pltpu.make_async_copy(pd_hbm.at[id_.at[pl.ds(0, _CHUNK)]],
                              bd[b], sg[b]).wait()

    def fire_write(i, b):
        off = lb + i * _CHUNK
        pltpu.async_copy(bs[b], g_hbm.at[pl.ds(off, _CHUNK), pl.ds(0, D)],
                         sw[b])
        pltpu.async_copy(bd[b], g_hbm.at[pl.ds(off, _CHUNK), pl.ds(D, D)],
                         sw[b])

    def drain_write(b):
        pltpu.make_async_copy(bs[b], g_hbm.at[pl.ds(0, _CHUNK), pl.ds(0, D)],
                              sw[b]).wait()
        pltpu.make_async_copy(bd[b], g_hbm.at[pl.ds(0, _CHUNK), pl.ds(D, D)],
                              sw[b]).wait()

    for b in range(_NBUF):
        fire(b, b)

    def body(i0):
        for b in range(_NBUF):
            i = i0 + b
            drain_gather(b)
            fire_write(i, b)
            drain_write(b)
            fire(i + _NBUF, b)

    pl.loop(0, ring - _NBUF, step=_NBUF)(body)

    for b in range(_NBUF):
        i = ring - _NBUF + b
        drain_gather(b)
        fire_write(i, b)
        drain_write(b)

    for i in range(ring, full):
        lo = i * _CHUNK
        cs = pltpu.async_copy(ps_hbm.at[ia.at[pl.ds(lo, _CHUNK)]],
                              bs[0], sg[0])
        cd = pltpu.async_copy(pd_hbm.at[id_.at[pl.ds(lo, _CHUNK)]],
                              bd[0], sg[0])
        cs.wait()
        cd.wait()
        off = lb + lo
        pltpu.sync_copy(bs[0], g_hbm.at[pl.ds(off, _CHUNK), pl.ds(0, D)])
        pltpu.sync_copy(bd[0], g_hbm.at[pl.ds(off, _CHUNK), pl.ds(D, D)])

    if tail:
        lo = full * _CHUNK
        pltpu.sync_copy(src_hbm.at[pl.ds(eb + lo, tail)], iat)
        pltpu.sync_copy(dst_hbm.at[pl.ds(eb + lo, tail)], idt)
        cs = pltpu.async_copy(ps_hbm.at[iat], bst, sgt)
        cd = pltpu.async_copy(pd_hbm.at[idt], bdt, swt)
        cs.wait()
        cd.wait()
        off = lb + lo
        pltpu.sync_copy(bst, g_hbm.at[pl.ds(off, tail), pl.ds(0, D)])
        pltpu.sync_copy(bdt, g_hbm.at[pl.ds(off, tail), pl.ds(D, D)])


@functools.lru_cache(maxsize=None)
def _sc_gather_kernel(half):
    tail = max(_G_TAIL[half], 16)

    @functools.partial(
        pl.kernel,
        mesh=_sc_mesh(),
        out_type=jax.ShapeDtypeStruct((_EHALF[half], D2), jnp.float32),
        scratch_types=[
            pltpu.VMEM((_EPW[half],), jnp.int32),
            pltpu.VMEM((_EPW[half],), jnp.int32),
            [pltpu.VMEM((_CHUNK, D), jnp.float32) for _ in range(_NBUF)],
            [pltpu.VMEM((_CHUNK, D), jnp.float32) for _ in range(_NBUF)],
            [pltpu.SemaphoreType.DMA for _ in range(_NBUF)],
            [pltpu.SemaphoreType.DMA for _ in range(_NBUF)],
            pltpu.VMEM((tail,), jnp.int32),
            pltpu.VMEM((tail,), jnp.int32),
            pltpu.VMEM((tail, D), jnp.float32),
            pltpu.VMEM((tail, D), jnp.float32),
            pltpu.SemaphoreType.DMA,
            pltpu.SemaphoreType.DMA,
        ],
        compiler_params=pltpu.CompilerParams(use_tc_tiling_on_sc=False),
    )
    def k(ps, pd, src, dst, g, *scratch):
        _sc_gather_body(half, ps, pd, src, dst, g, *scratch)

    return k


def _sc_gather(ps, pd, src, dst, half):
    return _sc_gather_kernel(half)(ps, pd, src, dst)


def _sc_scatter_body(half, e2_hbm, dst_hbm, zeros_hbm, out_hbm,
                     acc, rbs, ibs, si, sd, ss, ibt):
    eps = _EPS[half]
    full = _S_FULL[half]
    tail = _S_TAIL[half]
    ring = full - full % _NBUF_S

    cid = lax.axis_index("c")
    sid = lax.axis_index("s")
    base = cid * _NHALF
    r0 = sid * _RPS
    pltpu.sync_copy(zeros_hbm.at[pl.ds(r0, _RPS)], acc.at[pl.ds(r0, _RPS)])
    plsc.subcore_barrier()

    sb = sid * eps
    gb = _EOFF[half] + sb

    def fire(i, b):
        off = i * _S_CHUNK
        pltpu.async_copy(dst_hbm.at[pl.ds(gb + off, _S_CHUNK)], ibs[b], si[b])
        pltpu.async_copy(e2_hbm.at[pl.ds(sb + off, _S_CHUNK), pl.ds(0, D)],
                         rbs[b], sd[b])

    def remap(ib, kmax):
        for k in range(kmax):
            v = ib[pl.ds(k * 16, 16)]
            ok = (v >= base) & (v < base + _NHALF)
            ib[pl.ds(k * 16, 16)] = jnp.where(ok, v - base, _DUMP)

    def process(b):
        pltpu.make_async_copy(dst_hbm.at[pl.ds(0, _S_CHUNK)], ibs[b],
                              si[b]).wait()
        remap(ibs[b], _S_CHUNK // 16)
        pltpu.make_async_copy(e2_hbm.at[pl.ds(0, _S_CHUNK), pl.ds(0, D)],
                              rbs[b], sd[b]).wait()
        pltpu.async_copy(rbs[b], acc.at[ibs[b]], ss[b], add=True)

    def drain_scatter(b):
        pltpu.make_async_copy(rbs[b], acc.at[ibs[b]], ss[b]).wait()

    for b in range(_NBUF_S):
        fire(b, b)

    def body(i0):
        for b in range(_NBUF_S):
            i = i0 + b
            process(b)
            drain_scatter(b)
            fire(i + _NBUF_S, b)

    pl.loop(0, ring - _NBUF_S, step=_NBUF_S)(body)

    for b in range(_NBUF_S):
        process(b)
        drain_scatter(b)

    for i in range(ring, full):
        off = i * _S_CHUNK
        pltpu.sync_copy(dst_hbm.at[pl.ds(gb + off, _S_CHUNK)], ibs[0])
        remap(ibs[0], _S_CHUNK // 16)
        pltpu.sync_copy(e2_hbm.at[pl.ds(sb + off, _S_CHUNK), pl.ds(0, D)],
                        rbs[0])
        pltpu.sync_copy(rbs[0], acc.at[ibs[0]], add=True)

    if tail:
        off = full * _S_CHUNK
        pltpu.sync_copy(dst_hbm.at[pl.ds(gb + off, tail)], ibt)
        remap(ibt, tail // 16)
        pltpu.sync_copy(e2_hbm.at[pl.ds(sb + off, tail), pl.ds(0, D)],
                        rbs[0].at[pl.ds(0, tail)])
        pltpu.sync_copy(rbs[0].at[pl.ds(0, tail)], acc.at[ibt], add=True)

    plsc.subcore_barrier()
    pltpu.sync_copy(acc.at[pl.ds(r0, _RPS)],
                    out_hbm.at[pl.ds(cid * _ACC + r0, _RPS), pl.ds(0, D)])


@functools.lru_cache(maxsize=None)
def _sc_scatter_kernel(half):
    tail = max(_S_TAIL[half], 16)

    @functools.partial(
        pl.kernel,
        mesh=_sc_mesh(),
        out_type=jax.ShapeDtypeStruct((_NC * _ACC, D2), jnp.float32),
        scratch_types=[
            pltpu.VMEM_SHARED((_ACC, D), jnp.float32),
            [pltpu.VMEM((_S_CHUNK, D), jnp.float32) for _ in range(_NBUF_S)],
            [pltpu.VMEM((_S_CHUNK,), jnp.int32) for _ in range(_NBUF_S)],
            [pltpu.SemaphoreType.DMA for _ in range(_NBUF_S)],
            [pltpu.SemaphoreType.DMA for _ in range(_NBUF_S)],
            [pltpu.SemaphoreType.DMA for _ in range(_NBUF_S)],
            pltpu.VMEM((tail,), jnp.int32),
        ],
        compiler_params=pltpu.CompilerParams(use_tc_tiling_on_sc=False),
    )
    def k(e2, dst, zeros, out, *scratch):
        _sc_scatter_body(half, e2, dst, zeros, out, *scratch)

    return k


def _sc_scatter(e2, dst, zeros, half):
    return _sc_scatter_kernel(half)(e2, dst, zeros)


def kernel(features, edge_attr, params, edge_index):
    src = edge_index[0]
    dst = edge_index[1]

    fpad = features
    eapad = edge_attr

    enc_node = dict(params["enc_node"])
    enc_edge = dict(params["enc_edge"])

    def row(v):
        return v.reshape(1, -1)

    def prep_mlp(p):
        return {
            "W1": p["W1"], "b1": row(p["b1"]), "W2": p["W2"],
            "b2": row(p["b2"]), "g": row(p["g"]), "be": row(p["be"]),
        }

    enc_node_p = prep_mlp(enc_node)
    enc_edge_p = prep_mlp(enc_edge)

    eblks = []
    nblks = []
    for blk in params["blocks"]:
        ew = blk["edge"]
        eblks.append({
            "W1a": ew["W1"][0:D], "W1b": ew["W1"][D:2 * D],
            "W1c": ew["W1"][2 * D:3 * D], "b1": row(ew["b1"]),
            "W2": ew["W2"], "b2": row(ew["b2"]),
            "g": row(ew["g"]), "be": row(ew["be"]),
        })
        nw = blk["node"]
        nblks.append({
            "V1a": nw["W1"][0:D], "V1b": nw["W1"][D:2 * D],
            "b1": row(nw["b1"]), "W2": nw["W2"], "b2": row(nw["b2"]),
            "g": row(nw["g"]), "be": row(nw["be"]),
        })

    dec = {
        "W1": params["dec"]["W1"], "b1": row(params["dec"]["b1"]),
        "W2": params["dec"]["W2"], "b2": row(params["dec"]["b2"]),
    }

    zeros = jnp.zeros((_ACC, D), jnp.float32)

    x, ps, pd = _enc_node_call(fpad, enc_node_p, eblks[0]["W1a"],
                               eblks[0]["W1b"])
    eA = eB = None
    out = None
    for i in range(NB):
        gA = _sc_gather(ps, pd, src, dst, 0)
        gB = _sc_gather(ps, pd, src, dst, 1)
        if i == 0:
            eA = _edge0_call(eapad, gA, enc_edge_p, eblks[0], 0)
            aggA = _sc_scatter(eA, dst, zeros, 0)
            eB = _edge0_call(eapad, gB, enc_edge_p, eblks[0], 1)
        else:
            eA = _edge_call(eA, gA, eblks[i])
            aggA = _sc_scatter(eA, dst, zeros, 0)
            eB = _edge_call(eB, gB, eblks[i])
        aggB = _sc_scatter(eB, dst, zeros, 1)
        if i < NB - 1:
            x, ps, pd = _node_call(x, aggA, aggB, nblks[i],
                                   eblks[i + 1]["W1a"], eblks[i + 1]["W1b"])
        else:
            out = _node_dec_call(x, aggA, aggB, nblks[i], dec)
    return out
